# baseline (device time: 527683 ns/iter reference)
import jax
import jax.numpy as jnp
from jax import lax
from jax.experimental import pallas as pl
from jax.experimental.pallas import tpu as pltpu

N_DEV = 16
N_LAYERS = 3


def kernel(x, Win0, Wout0, Win1, Wout1, Win2, Wout2):
    b, d = x.shape
    hloc = Win0.shape[1]

    def body(x_ref, win0_ref, wout0_ref, win1_ref, wout1_ref, win2_ref,
             wout2_ref, out_ref, xb_ref, acc_ref, win_comm, wout_comm,
             win_send, win_recv, wout_send, wout_recv, credit_sem):
        my = lax.axis_index("i")
        left = lax.rem(my + N_DEV - 1, N_DEV)
        right = lax.rem(my + 1, N_DEV)

        barrier = pltpu.get_barrier_semaphore()
        for nbr in (left, right):
            pl.semaphore_signal(barrier, inc=1, device_id=(nbr,),
                                device_id_type=pl.DeviceIdType.MESH)
        pl.semaphore_wait(barrier, 2)

        xb_ref[...] = x_ref[...].astype(jnp.bfloat16)

        win_refs = (win0_ref, win1_ref, win2_ref)
        wout_refs = (wout0_ref, wout1_ref, wout2_ref)

        for l in range(N_LAYERS):
            acc_ref[...] = jnp.zeros_like(acc_ref)

            def step(h, carry, l=l):
                slot = lax.rem(h, 2)
                nxt = lax.rem(h + 1, 2)

                @pl.when(h == 0)
                def _():
                    win_comm[0] = win_refs[l][...].astype(jnp.bfloat16)
                    wout_comm[0] = wout_refs[l][...].astype(jnp.bfloat16)

                if l == 0:
                    @pl.when(h >= 1)
                    def _():
                        pl.semaphore_wait(credit_sem, 1)
                else:
                    pl.semaphore_wait(credit_sem, 1)

                copy_win = pltpu.make_async_remote_copy(
                    src_ref=win_comm.at[slot], dst_ref=win_comm.at[nxt],
                    send_sem=win_send.at[slot], recv_sem=win_recv.at[nxt],
                    device_id=(right,), device_id_type=pl.DeviceIdType.MESH)
                copy_wout = pltpu.make_async_remote_copy(
                    src_ref=wout_comm.at[slot], dst_ref=wout_comm.at[nxt],
                    send_sem=wout_send.at[slot], recv_sem=wout_recv.at[nxt],
                    device_id=(right,), device_id_type=pl.DeviceIdType.MESH)
                copy_win.start()
                copy_wout.start()

                h_act = jnp.dot(xb_ref[...], win_comm[slot],
                                preferred_element_type=jnp.float32)
                h_act = jnp.maximum(h_act, 0.0).astype(jnp.bfloat16)
                acc_ref[...] = acc_ref[...] + jnp.dot(
                    h_act, wout_comm[slot],
                    preferred_element_type=jnp.float32)

                copy_win.wait()
                copy_wout.wait()

                if l == N_LAYERS - 1:
                    @pl.when(h <= N_DEV - 2)
                    def _():
                        pl.semaphore_signal(
                            credit_sem, inc=1, device_id=(left,),
                            device_id_type=pl.DeviceIdType.MESH)
                else:
                    pl.semaphore_signal(
                        credit_sem, inc=1, device_id=(left,),
                        device_id_type=pl.DeviceIdType.MESH)
                return carry

            lax.fori_loop(0, N_DEV, step, None)

            if l < N_LAYERS - 1:
                xb_ref[...] = acc_ref[...].astype(jnp.bfloat16)
            else:
                out_ref[...] = acc_ref[...]

    return pl.pallas_call(
        body,
        out_shape=jax.ShapeDtypeStruct((b, d), jnp.float32),
        in_specs=[pl.BlockSpec(memory_space=pltpu.VMEM)] * 7,
        out_specs=pl.BlockSpec(memory_space=pltpu.VMEM),
        scratch_shapes=[
            pltpu.VMEM((b, d), jnp.bfloat16),
            pltpu.VMEM((b, d), jnp.float32),
            pltpu.VMEM((2, d, hloc), jnp.bfloat16),
            pltpu.VMEM((2, hloc, d), jnp.bfloat16),
            pltpu.SemaphoreType.DMA((2,)),
            pltpu.SemaphoreType.DMA((2,)),
            pltpu.SemaphoreType.DMA((2,)),
            pltpu.SemaphoreType.DMA((2,)),
            pltpu.SemaphoreType.REGULAR,
        ],
        compiler_params=pltpu.CompilerParams(collective_id=0),
    )(x, Win0, Wout0, Win1, Wout1, Win2, Wout2)


# device time: 201414 ns/iter; 2.6199x vs baseline; 2.6199x over previous
import jax
import jax.numpy as jnp
from jax import lax
from jax.experimental import pallas as pl
from jax.experimental.pallas import tpu as pltpu

N_DEV = 16
N_LAYERS = 3
K = 4
S_CW = 8
S_CCW = 7
CW_CONS = S_CW + 1
CCW_CONS = S_CCW + 1


def kernel(x, Win0, Wout0, Win1, Wout1, Win2, Wout2):
    b, d = x.shape
    hloc = Win0.shape[1]

    def body(x_ref, win0_ref, wout0_ref, win1_ref, wout1_ref, win2_ref,
             wout2_ref, out_ref, xb_ref, acc_ref,
             cw_win, cw_wout, ccw_win, ccw_wout,
             cw_win_send, cw_win_recv, cw_wout_send, cw_wout_recv,
             ccw_win_send, ccw_win_recv, ccw_wout_send, ccw_wout_recv,
             credit_cw, credit_ccw):
        my = lax.axis_index("i")
        left = lax.rem(my + N_DEV - 1, N_DEV)
        right = lax.rem(my + 1, N_DEV)

        barrier = pltpu.get_barrier_semaphore()
        for nbr in (left, right):
            pl.semaphore_signal(barrier, inc=1, device_id=(nbr,),
                                device_id_type=pl.DeviceIdType.MESH)
        pl.semaphore_wait(barrier, 2)

        xb_ref[...] = x_ref[...].astype(jnp.bfloat16)

        win_refs = (win0_ref, win1_ref, win2_ref)
        wout_refs = (wout0_ref, wout1_ref, wout2_ref)

        def start_pair(wbuf, obuf, wsend, wrecv, osend, orecv, slot, nxt,
                       dev):
            d1 = pltpu.make_async_remote_copy(
                src_ref=wbuf.at[slot], dst_ref=wbuf.at[nxt],
                send_sem=wsend.at[slot], recv_sem=wrecv.at[nxt],
                device_id=(dev,), device_id_type=pl.DeviceIdType.MESH)
            d2 = pltpu.make_async_remote_copy(
                src_ref=obuf.at[slot], dst_ref=obuf.at[nxt],
                send_sem=osend.at[slot], recv_sem=orecv.at[nxt],
                device_id=(dev,), device_id_type=pl.DeviceIdType.MESH)
            return d1, d2

        def wait_pair_recv(wbuf, obuf, wrecv, orecv, slot, dev):
            r1 = pltpu.make_async_remote_copy(
                src_ref=wbuf.at[slot], dst_ref=wbuf.at[slot],
                send_sem=wrecv.at[slot], recv_sem=wrecv.at[slot],
                device_id=(dev,), device_id_type=pl.DeviceIdType.MESH)
            r2 = pltpu.make_async_remote_copy(
                src_ref=obuf.at[slot], dst_ref=obuf.at[slot],
                send_sem=orecv.at[slot], recv_sem=orecv.at[slot],
                device_id=(dev,), device_id_type=pl.DeviceIdType.MESH)
            r1.wait_recv()
            r2.wait_recv()

        def contrib(win, wout):
            h = jnp.dot(xb_ref[...], win, preferred_element_type=jnp.float32)
            h = jnp.maximum(h, 0.0).astype(jnp.bfloat16)
            return jnp.dot(h, wout, preferred_element_type=jnp.float32)

        def signal(sem, dev):
            pl.semaphore_signal(sem, inc=1, device_id=(dev,),
                                device_id_type=pl.DeviceIdType.MESH)

        for l in range(N_LAYERS):
            Cb = CW_CONS * l
            Db = CCW_CONS * l
            acc_ref[...] = jnp.zeros_like(acc_ref)

            cs0, ds0 = Cb % K, Db % K
            wl = win_refs[l][...].astype(jnp.bfloat16)
            ol = wout_refs[l][...].astype(jnp.bfloat16)
            cw_win[cs0] = wl
            cw_wout[cs0] = ol
            ccw_win[ds0] = wl
            ccw_wout[ds0] = ol
            if l > 0:
                pl.semaphore_wait(credit_cw, 2)
                pl.semaphore_wait(credit_ccw, 2)
            c1, c2 = start_pair(cw_win, cw_wout, cw_win_send, cw_win_recv,
                                cw_wout_send, cw_wout_recv,
                                cs0, (Cb + 1) % K, right)
            w1, w2 = start_pair(ccw_win, ccw_wout, ccw_win_send, ccw_win_recv,
                                ccw_wout_send, ccw_wout_recv,
                                ds0, (Db + 1) % K, left)
            c1.start(), c2.start(), w1.start(), w2.start()
            acc_ref[...] = acc_ref[...] + contrib(cw_win[cs0], cw_wout[cs0])
            c1.wait_send(), c2.wait_send(), w1.wait_send(), w2.wait_send()
            signal(credit_cw, left)
            signal(credit_ccw, right)

            def step(s, carry, l=l, Cb=Cb, Db=Db):
                cslot = lax.rem(Cb + s, K)
                cnxt = lax.rem(Cb + s + 1, K)
                dslot = lax.rem(Db + s, K)
                dnxt = lax.rem(Db + s + 1, K)

                if l == 0:
                    @pl.when(s >= K - 1)
                    def _():
                        pl.semaphore_wait(credit_cw, 1)
                else:
                    pl.semaphore_wait(credit_cw, 1)
                wait_pair_recv(cw_win, cw_wout, cw_win_recv, cw_wout_recv,
                               cslot, left)
                f1, f2 = start_pair(cw_win, cw_wout, cw_win_send, cw_win_recv,
                                    cw_wout_send, cw_wout_recv,
                                    cslot, cnxt, right)
                f1.start(), f2.start()

                if l == 0:
                    @pl.when((s >= K - 1) & (s <= S_CCW - 1))
                    def _():
                        pl.semaphore_wait(credit_ccw, 1)
                else:
                    @pl.when(s <= S_CCW - 1)
                    def _():
                        pl.semaphore_wait(credit_ccw, 1)
                wait_pair_recv(ccw_win, ccw_wout, ccw_win_recv, ccw_wout_recv,
                               dslot, right)
                g1, g2 = start_pair(ccw_win, ccw_wout, ccw_win_send,
                                    ccw_win_recv, ccw_wout_send, ccw_wout_recv,
                                    dslot, dnxt, left)

                @pl.when(s <= S_CCW - 1)
                def _():
                    g1.start(), g2.start()

                acc_ref[...] = (acc_ref[...]
                                + contrib(cw_win[cslot], cw_wout[cslot])
                                + contrib(ccw_win[dslot], ccw_wout[dslot]))

                f1.wait_send(), f2.wait_send()
                if l < N_LAYERS - 1:
                    signal(credit_cw, left)
                else:
                    @pl.when(s <= 4)
                    def _():
                        signal(credit_cw, left)

                @pl.when(s <= S_CCW - 1)
                def _():
                    g1.wait_send(), g2.wait_send()
                if l < N_LAYERS - 1:
                    signal(credit_ccw, right)
                else:
                    @pl.when(s <= 3)
                    def _():
                        signal(credit_ccw, right)
                return carry

            lax.fori_loop(1, S_CW, step, None)

            cs8 = (Cb + S_CW) % K
            wait_pair_recv(cw_win, cw_wout, cw_win_recv, cw_wout_recv,
                           cs8, left)
            acc_ref[...] = acc_ref[...] + contrib(cw_win[cs8], cw_wout[cs8])
            if l < N_LAYERS - 1:
                signal(credit_cw, left)

            if l < N_LAYERS - 1:
                xb_ref[...] = acc_ref[...].astype(jnp.bfloat16)
            else:
                out_ref[...] = acc_ref[...]

    return pl.pallas_call(
        body,
        out_shape=jax.ShapeDtypeStruct((b, d), jnp.float32),
        in_specs=[pl.BlockSpec(memory_space=pltpu.VMEM)] * 7,
        out_specs=pl.BlockSpec(memory_space=pltpu.VMEM),
        scratch_shapes=[
            pltpu.VMEM((b, d), jnp.bfloat16),
            pltpu.VMEM((b, d), jnp.float32),
            pltpu.VMEM((K, d, hloc), jnp.bfloat16),
            pltpu.VMEM((K, hloc, d), jnp.bfloat16),
            pltpu.VMEM((K, d, hloc), jnp.bfloat16),
            pltpu.VMEM((K, hloc, d), jnp.bfloat16),
            pltpu.SemaphoreType.DMA((K,)),
            pltpu.SemaphoreType.DMA((K,)),
            pltpu.SemaphoreType.DMA((K,)),
            pltpu.SemaphoreType.DMA((K,)),
            pltpu.SemaphoreType.DMA((K,)),
            pltpu.SemaphoreType.DMA((K,)),
            pltpu.SemaphoreType.DMA((K,)),
            pltpu.SemaphoreType.DMA((K,)),
            pltpu.SemaphoreType.REGULAR,
            pltpu.SemaphoreType.REGULAR,
        ],
        compiler_params=pltpu.CompilerParams(collective_id=0),
    )(x, Win0, Wout0, Win1, Wout1, Win2, Wout2)


# device time: 181890 ns/iter; 2.9011x vs baseline; 1.1073x over previous
import jax
import jax.numpy as jnp
from jax import lax
from jax.experimental import pallas as pl
from jax.experimental.pallas import tpu as pltpu

N_DEV = 16
N_LAYERS = 3
K = 6
P = 2
H_CW = 8
H_CCW = 7
CW_CONS = P * (H_CW + 1)
CCW_CONS = P * (H_CCW + 1)
CW_SENDS = P * H_CW
CCW_SENDS = P * H_CCW


def kernel(x, Win0, Wout0, Win1, Wout1, Win2, Wout2):
    b, d = x.shape
    hloc = Win0.shape[1]
    hh = hloc // P

    def body(x_ref, win0_ref, wout0_ref, win1_ref, wout1_ref, win2_ref,
             wout2_ref, out_ref, xb_ref, acc_ref,
             cw_win, cw_wout, ccw_win, ccw_wout,
             cw_win_send, cw_win_recv, cw_wout_send, cw_wout_recv,
             ccw_win_send, ccw_win_recv, ccw_wout_send, ccw_wout_recv,
             credit_cw, credit_ccw):
        my = lax.axis_index("i")
        left = lax.rem(my + N_DEV - 1, N_DEV)
        right = lax.rem(my + 1, N_DEV)

        barrier = pltpu.get_barrier_semaphore()
        for nbr in (left, right):
            pl.semaphore_signal(barrier, inc=1, device_id=(nbr,),
                                device_id_type=pl.DeviceIdType.MESH)
        pl.semaphore_wait(barrier, 2)

        xb_ref[...] = x_ref[...].astype(jnp.bfloat16)

        win_refs = (win0_ref, win1_ref, win2_ref)
        wout_refs = (wout0_ref, wout1_ref, wout2_ref)

        def start_pair(wbuf, obuf, wsend, wrecv, osend, orecv, slot, dst,
                       dev):
            d1 = pltpu.make_async_remote_copy(
                src_ref=wbuf.at[slot], dst_ref=wbuf.at[dst],
                send_sem=wsend.at[slot], recv_sem=wrecv.at[dst],
                device_id=(dev,), device_id_type=pl.DeviceIdType.MESH)
            d2 = pltpu.make_async_remote_copy(
                src_ref=obuf.at[slot], dst_ref=obuf.at[dst],
                send_sem=osend.at[slot], recv_sem=orecv.at[dst],
                device_id=(dev,), device_id_type=pl.DeviceIdType.MESH)
            return d1, d2

        def wait_pair_recv(wbuf, obuf, wrecv, orecv, slot, dev):
            r1 = pltpu.make_async_remote_copy(
                src_ref=wbuf.at[slot], dst_ref=wbuf.at[slot],
                send_sem=wrecv.at[slot], recv_sem=wrecv.at[slot],
                device_id=(dev,), device_id_type=pl.DeviceIdType.MESH)
            r2 = pltpu.make_async_remote_copy(
                src_ref=obuf.at[slot], dst_ref=obuf.at[slot],
                send_sem=orecv.at[slot], recv_sem=orecv.at[slot],
                device_id=(dev,), device_id_type=pl.DeviceIdType.MESH)
            r1.wait_recv()
            r2.wait_recv()

        def contrib(win, wout):
            h = jnp.dot(xb_ref[...], win, preferred_element_type=jnp.float32)
            h = jnp.maximum(h, 0.0).astype(jnp.bfloat16)
            return jnp.dot(h, wout, preferred_element_type=jnp.float32)

        def signal(sem, dev):
            pl.semaphore_signal(sem, inc=1, device_id=(dev,),
                                device_id_type=pl.DeviceIdType.MESH)

        def cw_start(slot, dst):
            return start_pair(cw_win, cw_wout, cw_win_send, cw_win_recv,
                              cw_wout_send, cw_wout_recv, slot, dst, right)

        def ccw_start(slot, dst):
            return start_pair(ccw_win, ccw_wout, ccw_win_send, ccw_win_recv,
                              ccw_wout_send, ccw_wout_recv, slot, dst, left)

        for l in range(N_LAYERS):
            Cb = CW_CONS * l
            Db = CCW_CONS * l

            inj = []
            for q in range(P):
                wq = win_refs[l][:, q * hh:(q + 1) * hh].astype(jnp.bfloat16)
                oq = wout_refs[l][q * hh:(q + 1) * hh, :].astype(jnp.bfloat16)
                cw_win[(Cb + q) % K] = wq
                cw_wout[(Cb + q) % K] = oq
                ccw_win[(Db + q) % K] = wq
                ccw_wout[(Db + q) % K] = oq
                if l > 0:
                    pl.semaphore_wait(credit_cw, 3 if q == 0 else 1)
                    pl.semaphore_wait(credit_ccw, 3 if q == 0 else 1)
                d1, d2 = cw_start((Cb + q) % K, (Cb + q + P) % K)
                d3, d4 = ccw_start((Db + q) % K, (Db + q + P) % K)
                d1.start(), d2.start(), d3.start(), d4.start()
                inj += [d1, d2, d3, d4]

            acc_ref[...] = contrib(win_refs[l][...].astype(jnp.bfloat16),
                                   wout_refs[l][...].astype(jnp.bfloat16))

            for dsc in inj:
                dsc.wait_send()
            for _ in range(P):
                signal(credit_cw, left)
                signal(credit_ccw, right)

            def step(s, carry, l=l, Cb=Cb, Db=Db):
                cs = lax.rem(Cb + s, K)
                cd = lax.rem(Cb + s + P, K)
                ds = lax.rem(Db + s, K)
                dd = lax.rem(Db + s + P, K)

                if l == 0:
                    @pl.when(s >= K - P)
                    def _():
                        pl.semaphore_wait(credit_cw, 1)
                else:
                    pl.semaphore_wait(credit_cw, 1)
                wait_pair_recv(cw_win, cw_wout, cw_win_recv, cw_wout_recv,
                               cs, left)
                f1, f2 = cw_start(cs, cd)
                f1.start(), f2.start()

                if l == 0:
                    @pl.when((s >= K - P) & (s <= CCW_SENDS - 1))
                    def _():
                        pl.semaphore_wait(credit_ccw, 1)
                else:
                    @pl.when(s <= CCW_SENDS - 1)
                    def _():
                        pl.semaphore_wait(credit_ccw, 1)
                wait_pair_recv(ccw_win, ccw_wout, ccw_win_recv, ccw_wout_recv,
                               ds, right)
                g1, g2 = ccw_start(ds, dd)

                @pl.when(s <= CCW_SENDS - 1)
                def _():
                    g1.start(), g2.start()

                acc_ref[...] = (acc_ref[...]
                                + contrib(cw_win[cs], cw_wout[cs])
                                + contrib(ccw_win[ds], ccw_wout[ds]))

                f1.wait_send(), f2.wait_send()
                if l < N_LAYERS - 1:
                    signal(credit_cw, left)
                else:
                    @pl.when(s <= 11)
                    def _():
                        signal(credit_cw, left)

                @pl.when(s <= CCW_SENDS - 1)
                def _():
                    g1.wait_send(), g2.wait_send()
                if l < N_LAYERS - 1:
                    signal(credit_ccw, right)
                else:
                    @pl.when(s <= 9)
                    def _():
                        signal(credit_ccw, right)
                return carry

            lax.fori_loop(P, CW_SENDS, step, None)

            for s in (CW_SENDS, CW_SENDS + 1):
                cs = (Cb + s) % K
                wait_pair_recv(cw_win, cw_wout, cw_win_recv, cw_wout_recv,
                               cs, left)
                acc_ref[...] = acc_ref[...] + contrib(cw_win[cs],
                                                      cw_wout[cs])
                if l < N_LAYERS - 1:
                    signal(credit_cw, left)

            if l < N_LAYERS - 1:
                xb_ref[...] = acc_ref[...].astype(jnp.bfloat16)
            else:
                out_ref[...] = acc_ref[...]

    return pl.pallas_call(
        body,
        out_shape=jax.ShapeDtypeStruct((b, d), jnp.float32),
        in_specs=[pl.BlockSpec(memory_space=pltpu.VMEM)] * 7,
        out_specs=pl.BlockSpec(memory_space=pltpu.VMEM),
        scratch_shapes=[
            pltpu.VMEM((b, d), jnp.bfloat16),
            pltpu.VMEM((b, d), jnp.float32),
            pltpu.VMEM((K, d, hh), jnp.bfloat16),
            pltpu.VMEM((K, hh, d), jnp.bfloat16),
            pltpu.VMEM((K, d, hh), jnp.bfloat16),
            pltpu.VMEM((K, hh, d), jnp.bfloat16),
            pltpu.SemaphoreType.DMA((K,)),
            pltpu.SemaphoreType.DMA((K,)),
            pltpu.SemaphoreType.DMA((K,)),
            pltpu.SemaphoreType.DMA((K,)),
            pltpu.SemaphoreType.DMA((K,)),
            pltpu.SemaphoreType.DMA((K,)),
            pltpu.SemaphoreType.DMA((K,)),
            pltpu.SemaphoreType.DMA((K,)),
            pltpu.SemaphoreType.REGULAR,
            pltpu.SemaphoreType.REGULAR,
        ],
        compiler_params=pltpu.CompilerParams(collective_id=0),
    )(x, Win0, Wout0, Win1, Wout1, Win2, Wout2)


# device time: 176201 ns/iter; 2.9948x vs baseline; 1.0323x over previous
import jax
import jax.numpy as jnp
from jax import lax
from jax.experimental import pallas as pl
from jax.experimental.pallas import tpu as pltpu

N_DEV = 16
N_LAYERS = 3
K = 6
P = 2
H_CW = 8
H_CCW = 7
CW_CONS = P * (H_CW + 1)
CCW_CONS = P * (H_CCW + 1)
CW_SENDS = P * H_CW
CCW_SENDS = P * H_CCW


def kernel(x, Win0, Wout0, Win1, Wout1, Win2, Wout2):
    b, d = x.shape
    hloc = Win0.shape[1]
    hh = hloc // P

    def body(x_ref, win0_ref, wout0_ref, win1_ref, wout1_ref, win2_ref,
             wout2_ref, out_ref, xb_ref, acc_ref, cw_buf, ccw_buf,
             cw_send, cw_recv, ccw_send, ccw_recv, credit_cw, credit_ccw):
        my = lax.axis_index("i")
        left = lax.rem(my + N_DEV - 1, N_DEV)
        right = lax.rem(my + 1, N_DEV)

        barrier = pltpu.get_barrier_semaphore()
        for nbr in (left, right):
            pl.semaphore_signal(barrier, inc=1, device_id=(nbr,),
                                device_id_type=pl.DeviceIdType.MESH)
        pl.semaphore_wait(barrier, 2)

        xb_ref[...] = x_ref[...].astype(jnp.bfloat16)

        win_refs = (win0_ref, win1_ref, win2_ref)
        wout_refs = (wout0_ref, wout1_ref, wout2_ref)

        def fwd(buf, send, recv, slot, dst, dev):
            return pltpu.make_async_remote_copy(
                src_ref=buf.at[slot], dst_ref=buf.at[dst],
                send_sem=send.at[slot], recv_sem=recv.at[dst],
                device_id=(dev,), device_id_type=pl.DeviceIdType.MESH)

        def wait_recv(buf, recv, slot, dev):
            pltpu.make_async_remote_copy(
                src_ref=buf.at[slot], dst_ref=buf.at[slot],
                send_sem=recv.at[slot], recv_sem=recv.at[slot],
                device_id=(dev,), device_id_type=pl.DeviceIdType.MESH,
            ).wait_recv()

        def contrib(win, wout):
            h = jnp.dot(xb_ref[...], win, preferred_element_type=jnp.float32)
            h = jnp.maximum(h, 0.0).astype(jnp.bfloat16)
            return jnp.dot(h, wout, preferred_element_type=jnp.float32)

        def pair_contrib(packed):
            return contrib(packed[0:hh], packed[hh:P * hh])

        def signal(sem, dev):
            pl.semaphore_signal(sem, inc=1, device_id=(dev,),
                                device_id_type=pl.DeviceIdType.MESH)

        def inject(l):
            Cb = CW_CONS * l
            Db = CCW_CONS * l
            descs = []
            for q in range(P):
                wq = win_refs[l][:, q * hh:(q + 1) * hh].astype(jnp.bfloat16)
                oq = wout_refs[l][q * hh:(q + 1) * hh, :].astype(jnp.bfloat16)
                ci, di = (Cb + q) % K, (Db + q) % K
                cw_buf[ci, 0:hh, :] = wq
                cw_buf[ci, hh:P * hh, :] = oq
                ccw_buf[di, 0:hh, :] = wq
                ccw_buf[di, hh:P * hh, :] = oq
                if l > 0:
                    pl.semaphore_wait(credit_cw, 3 if q == 0 else 1)
                    pl.semaphore_wait(credit_ccw, 3 if q == 0 else 1)
                d1 = fwd(cw_buf, cw_send, cw_recv, ci, (Cb + q + P) % K,
                         right)
                d2 = fwd(ccw_buf, ccw_send, ccw_recv, di, (Db + q + P) % K,
                         left)
                d1.start(), d2.start()
                descs += [d1, d2]
            return descs

        inj = inject(0)

        for l in range(N_LAYERS):
            Cb = CW_CONS * l
            Db = CCW_CONS * l

            acc_ref[...] = contrib(win_refs[l][...].astype(jnp.bfloat16),
                                   wout_refs[l][...].astype(jnp.bfloat16))

            for dsc in inj:
                dsc.wait_send()
            for _ in range(P):
                signal(credit_cw, left)
                signal(credit_ccw, right)

            def step(s, carry, l=l, Cb=Cb, Db=Db):
                cs = lax.rem(Cb + s, K)
                cd = lax.rem(Cb + s + P, K)
                ds = lax.rem(Db + s, K)
                dd = lax.rem(Db + s + P, K)

                if l == 0:
                    @pl.when(s >= K - P)
                    def _():
                        pl.semaphore_wait(credit_cw, 1)
                else:
                    pl.semaphore_wait(credit_cw, 1)
                wait_recv(cw_buf, cw_recv, cs, left)
                f1 = fwd(cw_buf, cw_send, cw_recv, cs, cd, right)
                f1.start()

                if l == 0:
                    @pl.when((s >= K - P) & (s <= CCW_SENDS - 1))
                    def _():
                        pl.semaphore_wait(credit_ccw, 1)
                else:
                    @pl.when(s <= CCW_SENDS - 1)
                    def _():
                        pl.semaphore_wait(credit_ccw, 1)
                wait_recv(ccw_buf, ccw_recv, ds, right)
                g1 = fwd(ccw_buf, ccw_send, ccw_recv, ds, dd, left)

                @pl.when(s <= CCW_SENDS - 1)
                def _():
                    g1.start()

                acc_ref[...] = (acc_ref[...]
                                + pair_contrib(cw_buf[cs])
                                + pair_contrib(ccw_buf[ds]))

                f1.wait_send()
                if l < N_LAYERS - 1:
                    signal(credit_cw, left)
                else:
                    @pl.when(s <= 11)
                    def _():
                        signal(credit_cw, left)

                @pl.when(s <= CCW_SENDS - 1)
                def _():
                    g1.wait_send()
                if l < N_LAYERS - 1:
                    signal(credit_ccw, right)
                else:
                    @pl.when(s <= 9)
                    def _():
                        signal(credit_ccw, right)
                return carry

            lax.fori_loop(P, CW_SENDS, step, None)

            if l < N_LAYERS - 1:
                inj = inject(l + 1)

            for s in (CW_SENDS, CW_SENDS + 1):
                cs = (Cb + s) % K
                wait_recv(cw_buf, cw_recv, cs, left)
                acc_ref[...] = acc_ref[...] + pair_contrib(cw_buf[cs])
                if l < N_LAYERS - 1:
                    signal(credit_cw, left)

            if l < N_LAYERS - 1:
                xb_ref[...] = acc_ref[...].astype(jnp.bfloat16)
            else:
                out_ref[...] = acc_ref[...]

    return pl.pallas_call(
        body,
        out_shape=jax.ShapeDtypeStruct((b, d), jnp.float32),
        in_specs=[pl.BlockSpec(memory_space=pltpu.VMEM)] * 7,
        out_specs=pl.BlockSpec(memory_space=pltpu.VMEM),
        scratch_shapes=[
            pltpu.VMEM((b, d), jnp.bfloat16),
            pltpu.VMEM((b, d), jnp.float32),
            pltpu.VMEM((K, P * hh, d), jnp.bfloat16),
            pltpu.VMEM((K, P * hh, d), jnp.bfloat16),
            pltpu.SemaphoreType.DMA((K,)),
            pltpu.SemaphoreType.DMA((K,)),
            pltpu.SemaphoreType.DMA((K,)),
            pltpu.SemaphoreType.DMA((K,)),
            pltpu.SemaphoreType.REGULAR,
            pltpu.SemaphoreType.REGULAR,
        ],
        compiler_params=pltpu.CompilerParams(collective_id=0),
    )(x, Win0, Wout0, Win1, Wout1, Win2, Wout2)


# device time: 176194 ns/iter; 2.9949x vs baseline; 1.0000x over previous
import jax
import jax.numpy as jnp
from jax import lax
from jax.experimental import pallas as pl
from jax.experimental.pallas import tpu as pltpu

N_DEV = 16
N_LAYERS = 3
K = 8
P = 2
H_CW = 8
H_CCW = 7
CW_CONS = P * (H_CW + 1)
CCW_CONS = P * (H_CCW + 1)
CW_SENDS = P * H_CW
CCW_SENDS = P * H_CCW


def kernel(x, Win0, Wout0, Win1, Wout1, Win2, Wout2):
    b, d = x.shape
    hloc = Win0.shape[1]
    hh = hloc // P

    def body(x_ref, win0_ref, wout0_ref, win1_ref, wout1_ref, win2_ref,
             wout2_ref, out_ref, xb_ref, acc_ref, cw_buf, ccw_buf,
             cw_send, cw_recv, ccw_send, ccw_recv, credit_cw, credit_ccw):
        my = lax.axis_index("i")
        left = lax.rem(my + N_DEV - 1, N_DEV)
        right = lax.rem(my + 1, N_DEV)

        barrier = pltpu.get_barrier_semaphore()
        for nbr in (left, right):
            pl.semaphore_signal(barrier, inc=1, device_id=(nbr,),
                                device_id_type=pl.DeviceIdType.MESH)
        pl.semaphore_wait(barrier, 2)

        xb_ref[...] = x_ref[...].astype(jnp.bfloat16)

        win_refs = (win0_ref, win1_ref, win2_ref)
        wout_refs = (wout0_ref, wout1_ref, wout2_ref)

        def fwd(buf, send, recv, slot, dst, dev):
            return pltpu.make_async_remote_copy(
                src_ref=buf.at[slot], dst_ref=buf.at[dst],
                send_sem=send.at[slot], recv_sem=recv.at[dst],
                device_id=(dev,), device_id_type=pl.DeviceIdType.MESH)

        def wait_recv(buf, recv, slot, dev):
            pltpu.make_async_remote_copy(
                src_ref=buf.at[slot], dst_ref=buf.at[slot],
                send_sem=recv.at[slot], recv_sem=recv.at[slot],
                device_id=(dev,), device_id_type=pl.DeviceIdType.MESH,
            ).wait_recv()

        def contrib(win, wout):
            h = jnp.dot(xb_ref[...], win, preferred_element_type=jnp.float32)
            h = jnp.maximum(h, 0.0).astype(jnp.bfloat16)
            return jnp.dot(h, wout, preferred_element_type=jnp.float32)

        def pair_contrib(packed):
            return contrib(packed[0:hh], packed[hh:P * hh])

        def signal(sem, dev):
            pl.semaphore_signal(sem, inc=1, device_id=(dev,),
                                device_id_type=pl.DeviceIdType.MESH)

        def inject(l):
            Cb = CW_CONS * l
            Db = CCW_CONS * l
            descs = []
            for q in range(P):
                wq = win_refs[l][:, q * hh:(q + 1) * hh].astype(jnp.bfloat16)
                oq = wout_refs[l][q * hh:(q + 1) * hh, :].astype(jnp.bfloat16)
                ci, di = (Cb + q) % K, (Db + q) % K
                cw_buf[ci, 0:hh, :] = wq
                cw_buf[ci, hh:P * hh, :] = oq
                ccw_buf[di, 0:hh, :] = wq
                ccw_buf[di, hh:P * hh, :] = oq
                if l > 0:
                    pl.semaphore_wait(credit_cw, 3 if q == 0 else 1)
                    pl.semaphore_wait(credit_ccw, 3 if q == 0 else 1)
                d1 = fwd(cw_buf, cw_send, cw_recv, ci, (Cb + q + P) % K,
                         right)
                d2 = fwd(ccw_buf, ccw_send, ccw_recv, di, (Db + q + P) % K,
                         left)
                d1.start(), d2.start()
                descs += [d1, d2]
            return descs

        inj = inject(0)

        for l in range(N_LAYERS):
            Cb = CW_CONS * l
            Db = CCW_CONS * l

            acc_ref[...] = contrib(win_refs[l][...].astype(jnp.bfloat16),
                                   wout_refs[l][...].astype(jnp.bfloat16))

            for dsc in inj:
                dsc.wait_send()
            for _ in range(P):
                signal(credit_cw, left)
                signal(credit_ccw, right)

            def step(s, carry, l=l, Cb=Cb, Db=Db):
                cs = lax.rem(Cb + s, K)
                cd = lax.rem(Cb + s + P, K)
                ds = lax.rem(Db + s, K)
                dd = lax.rem(Db + s + P, K)

                if l == 0:
                    @pl.when(s >= K - P)
                    def _():
                        pl.semaphore_wait(credit_cw, 1)
                else:
                    pl.semaphore_wait(credit_cw, 1)
                wait_recv(cw_buf, cw_recv, cs, left)
                f1 = fwd(cw_buf, cw_send, cw_recv, cs, cd, right)
                f1.start()

                if l == 0:
                    @pl.when((s >= K - P) & (s <= CCW_SENDS - 1))
                    def _():
                        pl.semaphore_wait(credit_ccw, 1)
                else:
                    @pl.when(s <= CCW_SENDS - 1)
                    def _():
                        pl.semaphore_wait(credit_ccw, 1)
                wait_recv(ccw_buf, ccw_recv, ds, right)
                g1 = fwd(ccw_buf, ccw_send, ccw_recv, ds, dd, left)

                @pl.when(s <= CCW_SENDS - 1)
                def _():
                    g1.start()

                acc_ref[...] = (acc_ref[...]
                                + pair_contrib(cw_buf[cs])
                                + pair_contrib(ccw_buf[ds]))

                f1.wait_send()
                if l < N_LAYERS - 1:
                    signal(credit_cw, left)
                else:
                    @pl.when(s <= CW_SENDS - 1 - (K - P))
                    def _():
                        signal(credit_cw, left)

                @pl.when(s <= CCW_SENDS - 1)
                def _():
                    g1.wait_send()
                if l < N_LAYERS - 1:
                    signal(credit_ccw, right)
                else:
                    @pl.when(s <= CCW_SENDS - 1 - (K - P))
                    def _():
                        signal(credit_ccw, right)
                return carry

            lax.fori_loop(P, CW_SENDS, step, None)

            if l < N_LAYERS - 1:
                inj = inject(l + 1)

            for s in (CW_SENDS, CW_SENDS + 1):
                cs = (Cb + s) % K
                wait_recv(cw_buf, cw_recv, cs, left)
                acc_ref[...] = acc_ref[...] + pair_contrib(cw_buf[cs])
                if l < N_LAYERS - 1:
                    signal(credit_cw, left)

            if l < N_LAYERS - 1:
                xb_ref[...] = acc_ref[...].astype(jnp.bfloat16)
            else:
                out_ref[...] = acc_ref[...]

    return pl.pallas_call(
        body,
        out_shape=jax.ShapeDtypeStruct((b, d), jnp.float32),
        in_specs=[pl.BlockSpec(memory_space=pltpu.VMEM)] * 7,
        out_specs=pl.BlockSpec(memory_space=pltpu.VMEM),
        scratch_shapes=[
            pltpu.VMEM((b, d), jnp.bfloat16),
            pltpu.VMEM((b, d), jnp.float32),
            pltpu.VMEM((K, P * hh, d), jnp.bfloat16),
            pltpu.VMEM((K, P * hh, d), jnp.bfloat16),
            pltpu.SemaphoreType.DMA((K,)),
            pltpu.SemaphoreType.DMA((K,)),
            pltpu.SemaphoreType.DMA((K,)),
            pltpu.SemaphoreType.DMA((K,)),
            pltpu.SemaphoreType.REGULAR,
            pltpu.SemaphoreType.REGULAR,
        ],
        compiler_params=pltpu.CompilerParams(collective_id=0),
    )(x, Win0, Wout0, Win1, Wout1, Win2, Wout2)


# device time: 175406 ns/iter; 3.0084x vs baseline; 1.0045x over previous
import jax
import jax.numpy as jnp
from jax import lax
from jax.experimental import pallas as pl
from jax.experimental.pallas import tpu as pltpu

N_DEV = 16
N_LAYERS = 3
K = 8
P = 2
H_CW = 8
H_CCW = 7
CW_CONS = P * (H_CW + 1)
CCW_CONS = P * (H_CCW + 1)
CW_SENDS = P * H_CW
CCW_SENDS = P * H_CCW


def kernel(x, Win0, Wout0, Win1, Wout1, Win2, Wout2):
    b, d = x.shape
    hloc = Win0.shape[1]
    hh = hloc // P

    def body(x_ref, win0_ref, wout0_ref, win1_ref, wout1_ref, win2_ref,
             wout2_ref, out_ref, xb_ref, acc_ref, cw_buf, ccw_buf,
             cw_send, cw_recv, ccw_send, ccw_recv, credit_cw, credit_ccw):
        my = lax.axis_index("i")
        left = lax.rem(my + N_DEV - 1, N_DEV)
        right = lax.rem(my + 1, N_DEV)

        barrier = pltpu.get_barrier_semaphore()
        for nbr in (left, right):
            pl.semaphore_signal(barrier, inc=1, device_id=(nbr,),
                                device_id_type=pl.DeviceIdType.MESH)
        pl.semaphore_wait(barrier, 2)

        xb_ref[...] = x_ref[...].astype(jnp.bfloat16)

        win_refs = (win0_ref, win1_ref, win2_ref)
        wout_refs = (wout0_ref, wout1_ref, wout2_ref)

        def fwd(buf, send, recv, slot, dst, dev):
            return pltpu.make_async_remote_copy(
                src_ref=buf.at[slot], dst_ref=buf.at[dst],
                send_sem=send.at[slot], recv_sem=recv.at[dst],
                device_id=(dev,), device_id_type=pl.DeviceIdType.MESH)

        def wait_recv(buf, recv, slot, dev):
            pltpu.make_async_remote_copy(
                src_ref=buf.at[slot], dst_ref=buf.at[slot],
                send_sem=recv.at[slot], recv_sem=recv.at[slot],
                device_id=(dev,), device_id_type=pl.DeviceIdType.MESH,
            ).wait_recv()

        def contrib(win, wout):
            h = jnp.dot(xb_ref[...], win, preferred_element_type=jnp.float32)
            h = jnp.maximum(h, 0.0).astype(jnp.bfloat16)
            return jnp.dot(h, wout, preferred_element_type=jnp.float32)

        def pair_contrib(packed):
            return contrib(packed[0:hh], packed[hh:P * hh])

        def signal(sem, dev):
            pl.semaphore_signal(sem, inc=1, device_id=(dev,),
                                device_id_type=pl.DeviceIdType.MESH)

        def inject(l):
            Cb = CW_CONS * l
            Db = CCW_CONS * l
            descs = []
            for q in range(P):
                wq = win_refs[l][:, q * hh:(q + 1) * hh].astype(jnp.bfloat16)
                oq = wout_refs[l][q * hh:(q + 1) * hh, :].astype(jnp.bfloat16)
                ci, di = (Cb + q) % K, (Db + q) % K
                cw_buf[ci, 0:hh, :] = wq
                cw_buf[ci, hh:P * hh, :] = oq
                ccw_buf[di, 0:hh, :] = wq
                ccw_buf[di, hh:P * hh, :] = oq
                if l > 0:
                    pl.semaphore_wait(credit_cw, 3 if q == 0 else 1)
                    pl.semaphore_wait(credit_ccw, 3 if q == 0 else 1)
                d1 = fwd(cw_buf, cw_send, cw_recv, ci, (Cb + q + P) % K,
                         right)
                d2 = fwd(ccw_buf, ccw_send, ccw_recv, di, (Db + q + P) % K,
                         left)
                d1.start(), d2.start()
                descs += [d1, d2]
            return descs

        inj = inject(0)

        for l in range(N_LAYERS):
            Cb = CW_CONS * l
            Db = CCW_CONS * l

            acc_ref[...] = contrib(win_refs[l][...].astype(jnp.bfloat16),
                                   wout_refs[l][...].astype(jnp.bfloat16))

            for dsc in inj:
                dsc.wait_send()
            for _ in range(P):
                signal(credit_cw, left)
                signal(credit_ccw, right)

            def step(s, carry, l=l, Cb=Cb, Db=Db):
                cs = lax.rem(Cb + s, K)
                cd = lax.rem(Cb + s + P, K)
                ds = lax.rem(Db + s, K)
                dd = lax.rem(Db + s + P, K)

                if l == 0:
                    @pl.when(s >= K - P)
                    def _():
                        pl.semaphore_wait(credit_cw, 1)
                else:
                    pl.semaphore_wait(credit_cw, 1)
                wait_recv(cw_buf, cw_recv, cs, left)
                f1 = fwd(cw_buf, cw_send, cw_recv, cs, cd, right)
                f1.start()

                if l == 0:
                    @pl.when((s >= K - P) & (s <= CCW_SENDS - 1))
                    def _():
                        pl.semaphore_wait(credit_ccw, 1)
                else:
                    @pl.when(s <= CCW_SENDS - 1)
                    def _():
                        pl.semaphore_wait(credit_ccw, 1)
                wait_recv(ccw_buf, ccw_recv, ds, right)
                g1 = fwd(ccw_buf, ccw_send, ccw_recv, ds, dd, left)

                @pl.when(s <= CCW_SENDS - 1)
                def _():
                    g1.start()


                f1.wait_send()
                if l < N_LAYERS - 1:
                    signal(credit_cw, left)
                else:
                    @pl.when(s <= CW_SENDS - 1 - (K - P))
                    def _():
                        signal(credit_cw, left)

                @pl.when(s <= CCW_SENDS - 1)
                def _():
                    g1.wait_send()
                if l < N_LAYERS - 1:
                    signal(credit_ccw, right)
                else:
                    @pl.when(s <= CCW_SENDS - 1 - (K - P))
                    def _():
                        signal(credit_ccw, right)
                return carry

            lax.fori_loop(P, CW_SENDS, step, None)

            if l < N_LAYERS - 1:
                inj = inject(l + 1)

            for s in (CW_SENDS, CW_SENDS + 1):
                cs = (Cb + s) % K
                wait_recv(cw_buf, cw_recv, cs, left)
                if l < N_LAYERS - 1:
                    signal(credit_cw, left)

            if l < N_LAYERS - 1:
                xb_ref[...] = acc_ref[...].astype(jnp.bfloat16)
            else:
                out_ref[...] = acc_ref[...]

    return pl.pallas_call(
        body,
        out_shape=jax.ShapeDtypeStruct((b, d), jnp.float32),
        in_specs=[pl.BlockSpec(memory_space=pltpu.VMEM)] * 7,
        out_specs=pl.BlockSpec(memory_space=pltpu.VMEM),
        scratch_shapes=[
            pltpu.VMEM((b, d), jnp.bfloat16),
            pltpu.VMEM((b, d), jnp.float32),
            pltpu.VMEM((K, P * hh, d), jnp.bfloat16),
            pltpu.VMEM((K, P * hh, d), jnp.bfloat16),
            pltpu.SemaphoreType.DMA((K,)),
            pltpu.SemaphoreType.DMA((K,)),
            pltpu.SemaphoreType.DMA((K,)),
            pltpu.SemaphoreType.DMA((K,)),
            pltpu.SemaphoreType.REGULAR,
            pltpu.SemaphoreType.REGULAR,
        ],
        compiler_params=pltpu.CompilerParams(collective_id=0),
    )(x, Win0, Wout0, Win1, Wout1, Win2, Wout2)


# device time: 154757 ns/iter; 3.4098x vs baseline; 1.1334x over previous
import jax
import jax.numpy as jnp
from jax import lax
from jax.experimental import pallas as pl
from jax.experimental.pallas import tpu as pltpu

N_DEV = 16
N_LAYERS = 3
K = 8
P = 2
H_CW = 8
H_CCW = 7
CW_CONS = P * (H_CW + 1)
CCW_CONS = P * (H_CCW + 1)
CW_SENDS = P * H_CW
CCW_SENDS = P * H_CCW


def kernel(x, Win0, Wout0, Win1, Wout1, Win2, Wout2):
    b, d = x.shape
    hloc = Win0.shape[1]
    hh = hloc // P

    def body(x_ref, win0_ref, wout0_ref, win1_ref, wout1_ref, win2_ref,
             wout2_ref, out_ref, xb_ref, acc_ref, cw_buf, ccw_buf,
             cw_send, cw_recv, ccw_send, ccw_recv, credit_cw, credit_ccw):
        my = lax.axis_index("i")
        left = lax.rem(my + N_DEV - 1, N_DEV)
        right = lax.rem(my + 1, N_DEV)

        barrier = pltpu.get_barrier_semaphore()
        for nbr in (left, right):
            pl.semaphore_signal(barrier, inc=1, device_id=(nbr,),
                                device_id_type=pl.DeviceIdType.MESH)
        pl.semaphore_wait(barrier, 2)

        xb_ref[...] = x_ref[...].astype(jnp.bfloat16)

        win_refs = (win0_ref, win1_ref, win2_ref)
        wout_refs = (wout0_ref, wout1_ref, wout2_ref)

        def fwd(buf, send, recv, slot, dst, dev):
            return pltpu.make_async_remote_copy(
                src_ref=buf.at[slot], dst_ref=buf.at[dst],
                send_sem=send.at[slot], recv_sem=recv.at[dst],
                device_id=(dev,), device_id_type=pl.DeviceIdType.MESH)

        def wait_recv(buf, recv, slot, dev):
            pltpu.make_async_remote_copy(
                src_ref=buf.at[slot], dst_ref=buf.at[slot],
                send_sem=recv.at[slot], recv_sem=recv.at[slot],
                device_id=(dev,), device_id_type=pl.DeviceIdType.MESH,
            ).wait_recv()

        def contrib(win, wout):
            h = jnp.dot(xb_ref[...], win, preferred_element_type=jnp.float32)
            h = jnp.maximum(h, 0.0).astype(jnp.bfloat16)
            return jnp.dot(h, wout, preferred_element_type=jnp.float32)

        def pair_contrib(packed):
            return contrib(packed[0:hh], packed[hh:P * hh])

        def signal(sem, dev):
            pl.semaphore_signal(sem, inc=1, device_id=(dev,),
                                device_id_type=pl.DeviceIdType.MESH)

        def inject(l):
            Cb = CW_CONS * l
            Db = CCW_CONS * l
            descs = []
            for q in range(P):
                wq = win_refs[l][:, q * hh:(q + 1) * hh].astype(jnp.bfloat16)
                oq = wout_refs[l][q * hh:(q + 1) * hh, :].astype(jnp.bfloat16)
                ci, di = (Cb + q) % K, (Db + q) % K
                cw_buf[ci, 0:hh, :] = wq
                cw_buf[ci, hh:P * hh, :] = oq
                ccw_buf[di, 0:hh, :] = wq
                ccw_buf[di, hh:P * hh, :] = oq
                if l > 0:
                    pl.semaphore_wait(credit_cw, 3 if q == 0 else 1)
                    pl.semaphore_wait(credit_ccw, 3 if q == 0 else 1)
                d1 = fwd(cw_buf, cw_send, cw_recv, ci, (Cb + q + P) % K,
                         right)
                d2 = fwd(ccw_buf, ccw_send, ccw_recv, di, (Db + q + P) % K,
                         left)
                d1.start(), d2.start()
                descs += [d1, d2]
            return descs

        inj = inject(0)

        for l in range(N_LAYERS):
            Cb = CW_CONS * l
            Db = CCW_CONS * l

            acc_ref[...] = contrib(win_refs[l][...].astype(jnp.bfloat16),
                                   wout_refs[l][...].astype(jnp.bfloat16))

            for dsc in inj:
                dsc.wait_send()
            for _ in range(P):
                signal(credit_cw, left)
                signal(credit_ccw, right)

            def step(s, carry, l=l, Cb=Cb, Db=Db):
                cs = lax.rem(Cb + s, K)
                cd = lax.rem(Cb + s + P, K)
                ds = lax.rem(Db + s, K)
                dd = lax.rem(Db + s + P, K)

                if l == 0:
                    @pl.when(s >= K - P)
                    def _():
                        pl.semaphore_wait(credit_cw, 1)
                else:
                    pl.semaphore_wait(credit_cw, 1)
                wait_recv(cw_buf, cw_recv, cs, left)
                f1 = fwd(cw_buf, cw_send, cw_recv, cs, cd, right)
                f1.start()

                if l == 0:
                    @pl.when((s >= K - P) & (s <= CCW_SENDS - 1))
                    def _():
                        pl.semaphore_wait(credit_ccw, 1)
                else:
                    @pl.when(s <= CCW_SENDS - 1)
                    def _():
                        pl.semaphore_wait(credit_ccw, 1)
                wait_recv(ccw_buf, ccw_recv, ds, right)
                g1 = fwd(ccw_buf, ccw_send, ccw_recv, ds, dd, left)

                @pl.when(s <= CCW_SENDS - 1)
                def _():
                    g1.start()

                acc_ref[...] = (acc_ref[...]
                                + pair_contrib(cw_buf[cs])
                                + pair_contrib(ccw_buf[ds]))

                cp = lax.rem(Cb + s - 1, K)
                dp = lax.rem(Db + s - 1, K)

                @pl.when(s >= P + 1)
                def _():
                    fwd(cw_buf, cw_send, cw_recv, cp, cp, right).wait_send()
                if l < N_LAYERS - 1:
                    @pl.when(s >= P + 1)
                    def _():
                        signal(credit_cw, left)
                else:
                    @pl.when((s >= P + 1) & (s <= CW_SENDS - (K - P)))
                    def _():
                        signal(credit_cw, left)

                @pl.when((s >= P + 1) & (s <= CCW_SENDS))
                def _():
                    fwd(ccw_buf, ccw_send, ccw_recv, dp, dp, left).wait_send()
                if l < N_LAYERS - 1:
                    @pl.when(s >= P + 1)
                    def _():
                        signal(credit_ccw, right)

                    @pl.when(s == CW_SENDS - 1)
                    def _():
                        signal(credit_ccw, right)
                else:
                    @pl.when((s >= P + 1) & (s <= CCW_SENDS - (K - P)))
                    def _():
                        signal(credit_ccw, right)
                return carry

            lax.fori_loop(P, CW_SENDS, step, None)

            fp = (Cb + CW_SENDS - 1) % K
            fwd(cw_buf, cw_send, cw_recv, fp, fp, right).wait_send()
            if l < N_LAYERS - 1:
                signal(credit_cw, left)

            if l < N_LAYERS - 1:
                inj = inject(l + 1)

            for s in (CW_SENDS, CW_SENDS + 1):
                cs = (Cb + s) % K
                wait_recv(cw_buf, cw_recv, cs, left)
                acc_ref[...] = acc_ref[...] + pair_contrib(cw_buf[cs])
                if l < N_LAYERS - 1:
                    signal(credit_cw, left)

            if l < N_LAYERS - 1:
                xb_ref[...] = acc_ref[...].astype(jnp.bfloat16)
            else:
                out_ref[...] = acc_ref[...]

    return pl.pallas_call(
        body,
        out_shape=jax.ShapeDtypeStruct((b, d), jnp.float32),
        in_specs=[pl.BlockSpec(memory_space=pltpu.VMEM)] * 7,
        out_specs=pl.BlockSpec(memory_space=pltpu.VMEM),
        scratch_shapes=[
            pltpu.VMEM((b, d), jnp.bfloat16),
            pltpu.VMEM((b, d), jnp.float32),
            pltpu.VMEM((K, P * hh, d), jnp.bfloat16),
            pltpu.VMEM((K, P * hh, d), jnp.bfloat16),
            pltpu.SemaphoreType.DMA((K,)),
            pltpu.SemaphoreType.DMA((K,)),
            pltpu.SemaphoreType.DMA((K,)),
            pltpu.SemaphoreType.DMA((K,)),
            pltpu.SemaphoreType.REGULAR,
            pltpu.SemaphoreType.REGULAR,
        ],
        compiler_params=pltpu.CompilerParams(collective_id=0),
    )(x, Win0, Wout0, Win1, Wout1, Win2, Wout2)


# device time: 148405 ns/iter; 3.5557x vs baseline; 1.0428x over previous
import jax
import jax.numpy as jnp
from jax import lax
from jax.experimental import pallas as pl
from jax.experimental.pallas import tpu as pltpu

N_DEV = 16
N_LAYERS = 3
K = 8
P = 2
CONS = 2 * 8 + 1
SENDS = 15


def kernel(x, Win0, Wout0, Win1, Wout1, Win2, Wout2):
    b, d = x.shape
    hloc = Win0.shape[1]
    hh = hloc // P

    def body(x_ref, win0_ref, wout0_ref, win1_ref, wout1_ref, win2_ref,
             wout2_ref, out_ref, xb_ref, acc_ref, cw_buf, ccw_buf,
             cw_send, cw_recv, ccw_send, ccw_recv, credit_cw, credit_ccw):
        my = lax.axis_index("i")
        left = lax.rem(my + N_DEV - 1, N_DEV)
        right = lax.rem(my + 1, N_DEV)

        barrier = pltpu.get_barrier_semaphore()
        for nbr in (left, right):
            pl.semaphore_signal(barrier, inc=1, device_id=(nbr,),
                                device_id_type=pl.DeviceIdType.MESH)
        pl.semaphore_wait(barrier, 2)

        xb_ref[...] = x_ref[...].astype(jnp.bfloat16)

        win_refs = (win0_ref, win1_ref, win2_ref)
        wout_refs = (wout0_ref, wout1_ref, wout2_ref)

        def fwd(buf, send, recv, slot, dst, dev):
            return pltpu.make_async_remote_copy(
                src_ref=buf.at[slot], dst_ref=buf.at[dst],
                send_sem=send.at[slot], recv_sem=recv.at[dst],
                device_id=(dev,), device_id_type=pl.DeviceIdType.MESH)

        def wait_recv(buf, recv, slot, dev):
            pltpu.make_async_remote_copy(
                src_ref=buf.at[slot], dst_ref=buf.at[slot],
                send_sem=recv.at[slot], recv_sem=recv.at[slot],
                device_id=(dev,), device_id_type=pl.DeviceIdType.MESH,
            ).wait_recv()

        def contrib(win, wout):
            h = jnp.dot(xb_ref[...], win, preferred_element_type=jnp.float32)
            h = jnp.maximum(h, 0.0).astype(jnp.bfloat16)
            return jnp.dot(h, wout, preferred_element_type=jnp.float32)

        def pair_contrib(packed):
            return contrib(packed[0:hh], packed[hh:P * hh])

        def signal(sem, dev):
            pl.semaphore_signal(sem, inc=1, device_id=(dev,),
                                device_id_type=pl.DeviceIdType.MESH)

        def inject(l):
            Cb = CONS * l
            descs = []
            pieces = []
            for q in range(P):
                wq = win_refs[l][:, q * hh:(q + 1) * hh].astype(jnp.bfloat16)
                oq = wout_refs[l][q * hh:(q + 1) * hh, :].astype(jnp.bfloat16)
                pieces.append((wq, oq))
            for s in range(P):
                ci = (Cb + s) % K
                cwq, cwo = pieces[s]
                ccq, cco = pieces[P - 1 - s]
                cw_buf[ci, 0:hh, :] = cwq
                cw_buf[ci, hh:P * hh, :] = cwo
                ccw_buf[ci, 0:hh, :] = ccq
                ccw_buf[ci, hh:P * hh, :] = cco
                if l > 0:
                    pl.semaphore_wait(credit_cw, 3 if s == 0 else 1)
                    pl.semaphore_wait(credit_ccw, 3 if s == 0 else 1)
                d1 = fwd(cw_buf, cw_send, cw_recv, ci, (Cb + s + P) % K,
                         right)
                d2 = fwd(ccw_buf, ccw_send, ccw_recv, ci, (Cb + s + P) % K,
                         left)
                d1.start(), d2.start()
                descs += [d1, d2]
            return descs

        inj = inject(0)

        for l in range(N_LAYERS):
            Cb = CONS * l

            acc_ref[...] = contrib(win_refs[l][...].astype(jnp.bfloat16),
                                   wout_refs[l][...].astype(jnp.bfloat16))

            for dsc in inj:
                dsc.wait_send()
            for _ in range(P):
                signal(credit_cw, left)
                signal(credit_ccw, right)

            def step(s, carry, l=l, Cb=Cb):
                cs = lax.rem(Cb + s, K)
                cd = lax.rem(Cb + s + P, K)

                if l == 0:
                    @pl.when(s >= K - P)
                    def _():
                        pl.semaphore_wait(credit_cw, 1)

                    @pl.when(s >= K - P)
                    def _():
                        pl.semaphore_wait(credit_ccw, 1)
                else:
                    pl.semaphore_wait(credit_cw, 1)
                    pl.semaphore_wait(credit_ccw, 1)

                wait_recv(cw_buf, cw_recv, cs, left)
                fwd(cw_buf, cw_send, cw_recv, cs, cd, right).start()
                wait_recv(ccw_buf, ccw_recv, cs, right)
                fwd(ccw_buf, ccw_send, ccw_recv, cs, cd, left).start()

                acc_ref[...] = (acc_ref[...]
                                + pair_contrib(cw_buf[cs])
                                + pair_contrib(ccw_buf[cs]))

                cp = lax.rem(Cb + s - 1, K)

                @pl.when(s >= P + 1)
                def _():
                    fwd(cw_buf, cw_send, cw_recv, cp, cp, right).wait_send()
                    fwd(ccw_buf, ccw_send, ccw_recv, cp, cp, left).wait_send()
                if l < N_LAYERS - 1:
                    @pl.when(s >= P + 1)
                    def _():
                        signal(credit_cw, left)
                        signal(credit_ccw, right)
                else:
                    @pl.when((s >= P + 1) & (s <= SENDS - (K - P)))
                    def _():
                        signal(credit_cw, left)
                        signal(credit_ccw, right)
                return carry

            lax.fori_loop(P, SENDS, step, None)

            fp = (Cb + SENDS - 1) % K
            fwd(cw_buf, cw_send, cw_recv, fp, fp, right).wait_send()
            fwd(ccw_buf, ccw_send, ccw_recv, fp, fp, left).wait_send()
            if l < N_LAYERS - 1:
                signal(credit_cw, left)
                signal(credit_ccw, right)

            if l < N_LAYERS - 1:
                inj = inject(l + 1)

            for s in (SENDS, SENDS + 1):
                cs = (Cb + s) % K
                wait_recv(cw_buf, cw_recv, cs, left)
                wait_recv(ccw_buf, ccw_recv, cs, right)
                acc_ref[...] = (acc_ref[...]
                                + pair_contrib(cw_buf[cs])
                                + pair_contrib(ccw_buf[cs]))
                if l < N_LAYERS - 1:
                    signal(credit_cw, left)
                    signal(credit_ccw, right)

            if l < N_LAYERS - 1:
                xb_ref[...] = acc_ref[...].astype(jnp.bfloat16)
            else:
                out_ref[...] = acc_ref[...]

    return pl.pallas_call(
        body,
        out_shape=jax.ShapeDtypeStruct((b, d), jnp.float32),
        in_specs=[pl.BlockSpec(memory_space=pltpu.VMEM)] * 7,
        out_specs=pl.BlockSpec(memory_space=pltpu.VMEM),
        scratch_shapes=[
            pltpu.VMEM((b, d), jnp.bfloat16),
            pltpu.VMEM((b, d), jnp.float32),
            pltpu.VMEM((K, P * hh, d), jnp.bfloat16),
            pltpu.VMEM((K, P * hh, d), jnp.bfloat16),
            pltpu.SemaphoreType.DMA((K,)),
            pltpu.SemaphoreType.DMA((K,)),
            pltpu.SemaphoreType.DMA((K,)),
            pltpu.SemaphoreType.DMA((K,)),
            pltpu.SemaphoreType.REGULAR,
            pltpu.SemaphoreType.REGULAR,
        ],
        compiler_params=pltpu.CompilerParams(collective_id=0),
    )(x, Win0, Wout0, Win1, Wout1, Win2, Wout2)
